# baseline (speedup 1.0000x reference)
import numpy as np
import jax
import jax.numpy as jnp
from jax.experimental import pallas as pl
from jax.experimental.pallas import tpu as pltpu

_M = 20
_A_MIN, _A_MAX = 0.5, 5.0
_BOUNDS = np.linspace(0.0, 1.0, _M + 1)
_TWO_PI = float(2.0 * np.pi)

_IN1, _IN2, _IN3 = 0, 3, 6
_ID, _ITMIN, _IINVTR, _IINVRATE = 9, 10, 11, 12
_IINVY = 13
_IA_H, _IC_H = 16, 36
_IA_S, _IC_S = 56, 76
_IA_L, _IC_L = 96, 116
_NPARAMS = 160

_AC0, _AC1, _AC2, _AC3 = 1.5707288, -0.2121144, 0.0742610, -0.0187293
_CA1 = -(_TWO_PI ** 2) / 2.0
_CA2 = (_TWO_PI ** 4) / 24.0
_CA3 = -(_TWO_PI ** 6) / 720.0
_CA4 = (_TWO_PI ** 8) / 40320.0
_SB0 = _TWO_PI
_SB1 = -(_TWO_PI ** 3) / 6.0
_SB2 = (_TWO_PI ** 5) / 120.0
_SB3 = -(_TWO_PI ** 7) / 5040.0


import functools


@functools.lru_cache(maxsize=None)
def _perm_matrix(w):
    e = np.zeros((3 * w, 3 * w), np.float32)
    c, x = np.divmod(np.arange(3 * w), w)
    e[np.arange(3 * w), 3 * x + c] = 1.0
    return jnp.asarray(e, jnp.bfloat16)


def _normalize(v):
    return v / ((v[0] ** 2 + v[1] ** 2 + v[2] ** 2 + 1e-08) ** 0.5 + 1e-08)


def _orthogonal_vectors(normal1):
    normal1 = _normalize(normal1)
    d = -0.5 * (normal1[0] + normal1[1] + normal1[2])
    point = jnp.array([1.0, 0.0, 0.0], dtype=jnp.float32)
    t = -(point[0] * normal1[0] + point[1] * normal1[1] + point[2] * normal1[2] + d)
    normal2 = normal1 * t + point - 0.5
    normal2 = _normalize(normal2)
    normal2 = jnp.where(jnp.dot(normal2, point) < 0, -normal2, normal2)
    normal3 = jnp.cross(normal1, normal2)
    normal3 = _normalize(normal3)
    return normal1, normal2, normal3


def _table(u):
    a = _A_MIN + (_A_MAX - _A_MIN) * jax.nn.sigmoid(u)
    delta = 1.0 / _M
    prefix = jnp.concatenate([jnp.zeros((1,), jnp.float32), jnp.cumsum(a)[:-1]]) * delta
    c = prefix - a * jnp.asarray(_BOUNDS[:_M], jnp.float32)
    inv_y = 1.0 / jnp.sum(a * delta)
    return a, c, inv_y


def _pwl(params, s, base_a, base_c, inv_y):
    acc_a = params[0, base_a]
    acc_c = params[0, base_c]
    for i in range(1, _M):
        m = s >= float(_BOUNDS[i])
        acc_a = jnp.where(m, params[0, base_a + i], acc_a)
        acc_c = jnp.where(m, params[0, base_c + i], acc_c)
    return (acc_c + acc_a * s) * inv_y


def _body(params_ref, x_ref, perm_ref, out_ref):
    p = lambda i: params_ref[0, i]
    n10, n11, n12 = p(_IN1), p(_IN1 + 1), p(_IN1 + 2)
    n20, n21, n22 = p(_IN2), p(_IN2 + 1), p(_IN2 + 2)
    n30, n31, n32 = p(_IN3), p(_IN3 + 1), p(_IN3 + 2)
    d = p(_ID)

    R = x_ref[0, 0]
    G = x_ref[0, 1]
    B = x_ref[0, 2]

    t = -(n10 * R + n11 * G + n12 * B + d)
    v0 = n10 * t + R - 0.5
    v1 = n11 * t + G - 0.5
    v2 = n12 * t + B - 0.5
    dist = jnp.sqrt(v0 * v0 + v1 * v1 + v2 * v2 + 1e-08)
    inv = 1.0 / (dist + 1e-08)
    cosv = jnp.clip((v0 * n20 + v1 * n21 + v2 * n22) * inv, -1 + 1e-05, 1 - 1e-05)
    sinv = jnp.clip((v0 * n30 + v1 * n31 + v2 * n32) * inv, -1 + 1e-05, 1 - 1e-05)

    xa = jnp.abs(cosv)
    r = jnp.sqrt(1.0 - xa) * (_AC0 + xa * (_AC1 + xa * (_AC2 + xa * _AC3)))
    u = r * (1.0 / _TWO_PI)
    u = jnp.where(cosv < 0, 0.5 - u, u)
    u = jnp.where(sinv < 0, 1.0 - u, u)

    ph = _pwl(params_ref, u, _IA_H, _IC_H, p(_IINVY))

    q4 = jnp.floor(4.0 * ph + 0.5)
    z = ph - 0.25 * q4
    zz = z * z
    cz = 1.0 + zz * (_CA1 + zz * (_CA2 + zz * (_CA3 + zz * _CA4)))
    sz = z * (_SB0 + zz * (_SB1 + zz * (_SB2 + zz * _SB3)))
    qm = q4 - 4.0 * jnp.floor(q4 * 0.25)
    m1 = qm == 1.0
    m2 = qm == 2.0
    m3 = qm == 3.0
    cosp = jnp.where(m1, -sz, jnp.where(m2, -cz, jnp.where(m3, sz, cz)))
    sinp = jnp.where(m1, cz, jnp.where(m2, -sz, jnp.where(m3, -cz, sz)))

    dn = dist * p(_IINVRATE)
    ds = _pwl(params_ref, dn, _IA_S, _IC_S, p(_IINVY + 1))
    h1 = jnp.clip(ds * cosp, -1.0, 1.0)
    h2 = jnp.clip(ds * sinp, -1.0, 1.0)

    tin = (t - p(_ITMIN)) * p(_IINVTR)
    tt = _pwl(params_ref, tin, _IA_L, _IC_L, p(_IINVY + 2))

    x = jnp.concatenate([tt, h1, h2], axis=1).astype(jnp.bfloat16)
    out_ref[0] = jax.lax.dot_general(
        x, perm_ref[...], (((1,), (0,)), ((), ())),
        preferred_element_type=jnp.float32)


def kernel(img, normal_vector_bias, u_hue, u_sat, u_lum):
    nb = jnp.clip(normal_vector_bias, -0.9, 0.9)
    normal = jnp.array([1.0, 1.0, 1.0], dtype=jnp.float32) + nb
    n1, n2, n3 = _orthogonal_vectors(_normalize(normal))
    d = -0.5 * (n1[0] + n1[1] + n1[2])
    t_min = n1[0] + n1[1] + n1[2] + d
    t_max = -(n1[0] + n1[1] + n1[2] + d)
    inv_tr = 1.0 / (t_max - t_min + 1e-06)

    points = jnp.array(
        [[1, 0, 0], [1, 1, 0], [0, 1, 0], [0, 1, 1], [0, 0, 1], [1, 0, 1], [1, 0, 0]],
        dtype=jnp.float32)
    tp = -(points @ n1 + d)
    point_flat = n1[None, :] * tp[:, None] + points - 0.5
    ranges = jnp.sqrt(jnp.sum(point_flat ** 2, axis=1) + 1e-08)
    rate = jnp.max(ranges)
    inv_rate = jnp.where(rate == 0, 0.0, 1.0 / (rate + 1e-08))

    a_h, c_h, iy_h = _table(u_hue)
    a_s, c_s, iy_s = _table(u_sat)
    a_l, c_l, iy_l = _table(u_lum)

    params = jnp.zeros((_NPARAMS,), jnp.float32)
    params = params.at[_IN1:_IN1 + 3].set(n1)
    params = params.at[_IN2:_IN2 + 3].set(n2)
    params = params.at[_IN3:_IN3 + 3].set(n3)
    params = params.at[_ID].set(d)
    params = params.at[_ITMIN].set(t_min)
    params = params.at[_IINVTR].set(inv_tr)
    params = params.at[_IINVRATE].set(inv_rate)
    params = params.at[_IINVY].set(iy_h)
    params = params.at[_IINVY + 1].set(iy_s)
    params = params.at[_IINVY + 2].set(iy_l)
    params = params.at[_IA_H:_IA_H + _M].set(a_h)
    params = params.at[_IC_H:_IC_H + _M].set(c_h)
    params = params.at[_IA_S:_IA_S + _M].set(a_s)
    params = params.at[_IC_S:_IC_S + _M].set(c_s)
    params = params.at[_IA_L:_IA_L + _M].set(a_l)
    params = params.at[_IC_L:_IC_L + _M].set(c_l)
    params = params.reshape(1, _NPARAMS)

    Bb, C, H, W = img.shape
    BR = 256
    grid = (Bb, H // BR)
    out = pl.pallas_call(
        _body,
        grid=grid,
        in_specs=[
            pl.BlockSpec(memory_space=pltpu.SMEM),
            pl.BlockSpec((1, C, BR, W), lambda b, r: (b, 0, r, 0)),
            pl.BlockSpec((3 * W, 3 * W), lambda b, r: (0, 0)),
        ],
        out_specs=pl.BlockSpec((1, BR, 3 * W), lambda b, r: (b, r, 0)),
        out_shape=jax.ShapeDtypeStruct((Bb, H, 3 * W), jnp.float32),
        compiler_params=pltpu.CompilerParams(
            dimension_semantics=("parallel", "parallel")),
    )(params, img, _perm_matrix(W))
    return out.reshape(Bb, H, W, 3)


# speedup vs baseline: 1.5869x; 1.5869x over previous
import numpy as np
import jax
import jax.numpy as jnp
from jax.experimental import pallas as pl
from jax.experimental.pallas import tpu as pltpu

_M = 20
_A_MIN, _A_MAX = 0.5, 5.0
_BOUNDS = np.linspace(0.0, 1.0, _M + 1)
_TWO_PI = float(2.0 * np.pi)

_IN1, _IN2, _IN3 = 0, 3, 6
_ID, _ITMIN, _IINVTR, _IINVRATE = 9, 10, 11, 12
_IINVY = 13
_IA_H, _IC_H = 16, 36
_IA_S, _IC_S = 56, 76
_IA_L, _IC_L = 96, 116
_NPARAMS = 160

_AC0, _AC1, _AC2, _AC3 = 1.5707288, -0.2121144, 0.0742610, -0.0187293
_CA1 = -(_TWO_PI ** 2) / 2.0
_CA2 = (_TWO_PI ** 4) / 24.0
_CA3 = -(_TWO_PI ** 6) / 720.0
_CA4 = (_TWO_PI ** 8) / 40320.0
_SB0 = _TWO_PI
_SB1 = -(_TWO_PI ** 3) / 6.0
_SB2 = (_TWO_PI ** 5) / 120.0
_SB3 = -(_TWO_PI ** 7) / 5040.0


import functools


@functools.lru_cache(maxsize=None)
def _perm_matrix(w):
    e = np.zeros((3 * w, 3 * w), np.float32)
    c, x = np.divmod(np.arange(3 * w), w)
    e[np.arange(3 * w), 3 * x + c] = 1.0
    return jnp.asarray(e, jnp.bfloat16)


def _normalize(v):
    return v / ((v[0] ** 2 + v[1] ** 2 + v[2] ** 2 + 1e-08) ** 0.5 + 1e-08)


def _orthogonal_vectors(normal1):
    normal1 = _normalize(normal1)
    d = -0.5 * (normal1[0] + normal1[1] + normal1[2])
    point = jnp.array([1.0, 0.0, 0.0], dtype=jnp.float32)
    t = -(point[0] * normal1[0] + point[1] * normal1[1] + point[2] * normal1[2] + d)
    normal2 = normal1 * t + point - 0.5
    normal2 = _normalize(normal2)
    normal2 = jnp.where(jnp.dot(normal2, point) < 0, -normal2, normal2)
    normal3 = jnp.cross(normal1, normal2)
    normal3 = _normalize(normal3)
    return normal1, normal2, normal3


def _table(u):
    a = _A_MIN + (_A_MAX - _A_MIN) * jax.nn.sigmoid(u)
    delta = 1.0 / _M
    prefix = jnp.concatenate([jnp.zeros((1,), jnp.float32), jnp.cumsum(a)[:-1]]) * delta
    c = prefix - a * jnp.asarray(_BOUNDS[:_M], jnp.float32)
    inv_y = 1.0 / jnp.sum(a * delta)
    return a, c, inv_y


def _pwl(params, s, base_a, base_c, inv_y):
    acc_a = params[0, base_a]
    acc_c = params[0, base_c]
    for i in range(1, _M):
        m = s >= float(_BOUNDS[i])
        acc_a = jnp.where(m, params[0, base_a + i], acc_a)
        acc_c = jnp.where(m, params[0, base_c + i], acc_c)
    return (acc_c + acc_a * s) * inv_y


def _body(params_ref, x_ref, out_ref):
    p = lambda i: params_ref[0, i]
    n10, n11, n12 = p(_IN1), p(_IN1 + 1), p(_IN1 + 2)
    n20, n21, n22 = p(_IN2), p(_IN2 + 1), p(_IN2 + 2)
    n30, n31, n32 = p(_IN3), p(_IN3 + 1), p(_IN3 + 2)
    d = p(_ID)

    R = x_ref[0, 0]
    G = x_ref[0, 1]
    B = x_ref[0, 2]

    t = -(n10 * R + n11 * G + n12 * B + d)
    v0 = n10 * t + R - 0.5
    v1 = n11 * t + G - 0.5
    v2 = n12 * t + B - 0.5
    dist = jnp.sqrt(v0 * v0 + v1 * v1 + v2 * v2 + 1e-08)
    inv = 1.0 / (dist + 1e-08)
    cosv = jnp.clip((v0 * n20 + v1 * n21 + v2 * n22) * inv, -1 + 1e-05, 1 - 1e-05)
    sinv = jnp.clip((v0 * n30 + v1 * n31 + v2 * n32) * inv, -1 + 1e-05, 1 - 1e-05)

    xa = jnp.abs(cosv)
    r = jnp.sqrt(1.0 - xa) * (_AC0 + xa * (_AC1 + xa * (_AC2 + xa * _AC3)))
    u = r * (1.0 / _TWO_PI)
    u = jnp.where(cosv < 0, 0.5 - u, u)
    u = jnp.where(sinv < 0, 1.0 - u, u)

    ph = _pwl(params_ref, u, _IA_H, _IC_H, p(_IINVY))

    q4 = jnp.floor(4.0 * ph + 0.5)
    z = ph - 0.25 * q4
    zz = z * z
    cz = 1.0 + zz * (_CA1 + zz * (_CA2 + zz * (_CA3 + zz * _CA4)))
    sz = z * (_SB0 + zz * (_SB1 + zz * (_SB2 + zz * _SB3)))
    qm = q4 - 4.0 * jnp.floor(q4 * 0.25)
    m1 = qm == 1.0
    m2 = qm == 2.0
    m3 = qm == 3.0
    cosp = jnp.where(m1, -sz, jnp.where(m2, -cz, jnp.where(m3, sz, cz)))
    sinp = jnp.where(m1, cz, jnp.where(m2, -sz, jnp.where(m3, -cz, sz)))

    dn = dist * p(_IINVRATE)
    ds = _pwl(params_ref, dn, _IA_S, _IC_S, p(_IINVY + 1))
    out_ref[0, 1] = jnp.clip(ds * cosp, -1.0, 1.0)
    out_ref[0, 2] = jnp.clip(ds * sinp, -1.0, 1.0)

    tin = (t - p(_ITMIN)) * p(_IINVTR)
    out_ref[0, 0] = _pwl(params_ref, tin, _IA_L, _IC_L, p(_IINVY + 2))


def kernel(img, normal_vector_bias, u_hue, u_sat, u_lum):
    nb = jnp.clip(normal_vector_bias, -0.9, 0.9)
    normal = jnp.array([1.0, 1.0, 1.0], dtype=jnp.float32) + nb
    n1, n2, n3 = _orthogonal_vectors(_normalize(normal))
    d = -0.5 * (n1[0] + n1[1] + n1[2])
    t_min = n1[0] + n1[1] + n1[2] + d
    t_max = -(n1[0] + n1[1] + n1[2] + d)
    inv_tr = 1.0 / (t_max - t_min + 1e-06)

    points = jnp.array(
        [[1, 0, 0], [1, 1, 0], [0, 1, 0], [0, 1, 1], [0, 0, 1], [1, 0, 1], [1, 0, 0]],
        dtype=jnp.float32)
    tp = -(points @ n1 + d)
    point_flat = n1[None, :] * tp[:, None] + points - 0.5
    ranges = jnp.sqrt(jnp.sum(point_flat ** 2, axis=1) + 1e-08)
    rate = jnp.max(ranges)
    inv_rate = jnp.where(rate == 0, 0.0, 1.0 / (rate + 1e-08))

    a_h, c_h, iy_h = _table(u_hue)
    a_s, c_s, iy_s = _table(u_sat)
    a_l, c_l, iy_l = _table(u_lum)

    params = jnp.zeros((_NPARAMS,), jnp.float32)
    params = params.at[_IN1:_IN1 + 3].set(n1)
    params = params.at[_IN2:_IN2 + 3].set(n2)
    params = params.at[_IN3:_IN3 + 3].set(n3)
    params = params.at[_ID].set(d)
    params = params.at[_ITMIN].set(t_min)
    params = params.at[_IINVTR].set(inv_tr)
    params = params.at[_IINVRATE].set(inv_rate)
    params = params.at[_IINVY].set(iy_h)
    params = params.at[_IINVY + 1].set(iy_s)
    params = params.at[_IINVY + 2].set(iy_l)
    params = params.at[_IA_H:_IA_H + _M].set(a_h)
    params = params.at[_IC_H:_IC_H + _M].set(c_h)
    params = params.at[_IA_S:_IA_S + _M].set(a_s)
    params = params.at[_IC_S:_IC_S + _M].set(c_s)
    params = params.at[_IA_L:_IA_L + _M].set(a_l)
    params = params.at[_IC_L:_IC_L + _M].set(c_l)
    params = params.reshape(1, _NPARAMS)

    Bb, C, H, W = img.shape
    BR = 256
    grid = (Bb, H // BR)
    out = pl.pallas_call(
        _body,
        grid=grid,
        in_specs=[
            pl.BlockSpec(memory_space=pltpu.SMEM),
            pl.BlockSpec((1, C, BR, W), lambda b, r: (b, 0, r, 0)),
        ],
        out_specs=pl.BlockSpec((1, 3, BR, W), lambda b, r: (b, 0, r, 0)),
        out_shape=jax.ShapeDtypeStruct((Bb, 3, H, W), jnp.float32),
        compiler_params=pltpu.CompilerParams(
            dimension_semantics=("parallel", "parallel")),
    )(params, img)
    return jnp.transpose(out, (0, 2, 3, 1))
